# initial kernel scaffold (unmeasured)
import jax
import jax.numpy as jnp
from jax import lax
from jax.experimental import pallas as pl
from jax.experimental.pallas import tpu as pltpu


def kernel(
    x,
):
    def body(*refs):
        pass

    out_shape = jax.ShapeDtypeStruct(..., jnp.float32)
    return pl.pallas_call(body, out_shape=out_shape)(...)


# baseline (device time: 55520 ns/iter reference)
import jax
import jax.numpy as jnp
from jax import lax
from jax.experimental import pallas as pl
from jax.experimental.pallas import tpu as pltpu


def kernel(x):
    m, n = x.shape
    n_half = n // 2

    def body(x_ref, out_ref, send_sem, recv_sem):
        my_x = lax.axis_index("x")
        my_y = lax.axis_index("y")
        peer_y = 1 - my_y

        barrier_sem = pltpu.get_barrier_semaphore()
        pl.semaphore_signal(
            barrier_sem, inc=1,
            device_id=(my_x, peer_y), device_id_type=pl.DeviceIdType.MESH,
        )
        pl.semaphore_wait(barrier_sem, 1)

        out_ref[pl.ds(my_y * m, m), :] = x_ref[:, pl.ds(my_y * n_half, n_half)]

        send_d = pltpu.make_async_remote_copy(
            src_ref=x_ref.at[:, pl.ds(peer_y * n_half, n_half)],
            dst_ref=out_ref.at[pl.ds(my_y * m, m), :],
            send_sem=send_sem,
            recv_sem=recv_sem,
            device_id=(my_x, peer_y),
            device_id_type=pl.DeviceIdType.MESH,
        )
        recv_d = pltpu.make_async_remote_copy(
            src_ref=x_ref.at[:, pl.ds(peer_y * n_half, n_half)],
            dst_ref=out_ref.at[pl.ds(peer_y * m, m), :],
            send_sem=send_sem,
            recv_sem=recv_sem,
            device_id=(my_x, peer_y),
            device_id_type=pl.DeviceIdType.MESH,
        )
        send_d.start()
        send_d.wait_send()
        recv_d.wait_recv()

    return pl.pallas_call(
        body,
        out_shape=jax.ShapeDtypeStruct((2 * m, n_half), x.dtype),
        in_specs=[pl.BlockSpec(memory_space=pltpu.VMEM)],
        out_specs=pl.BlockSpec(memory_space=pltpu.VMEM),
        scratch_shapes=[
            pltpu.SemaphoreType.DMA,
            pltpu.SemaphoreType.DMA,
        ],
        compiler_params=pltpu.CompilerParams(collective_id=0),
    )(x)


# device time: 37221 ns/iter; 1.4916x vs baseline; 1.4916x over previous
import jax
import jax.numpy as jnp
from jax import lax
from jax.experimental import pallas as pl
from jax.experimental.pallas import tpu as pltpu

N_CHUNKS = 16


def kernel(x):
    m, n = x.shape
    n_half = n // 2
    h = m // 2
    hc = h // N_CHUNKS

    def body(x_ref, out_ref, p1_send, p1_recv, p2_send, p2_recv):
        my_x = lax.axis_index("x")
        my_y = lax.axis_index("y")
        peer_y = 1 - my_y
        peer_x = 1 - my_x

        barrier_sem = pltpu.get_barrier_semaphore()
        for dev in [(my_x, peer_y), (peer_x, my_y)]:
            pl.semaphore_signal(
                barrier_sem, inc=1,
                device_id=dev, device_id_type=pl.DeviceIdType.MESH,
            )
        pl.semaphore_wait(barrier_sem, 2)

        p1_base = peer_y * m + my_x * h

        p1 = []
        for c in range(N_CHUNKS):
            d = pltpu.make_async_remote_copy(
                src_ref=x_ref.at[
                    pl.ds(my_x * h + c * hc, hc),
                    pl.ds(peer_y * n_half, n_half),
                ],
                dst_ref=out_ref.at[pl.ds(my_y * m + my_x * h + c * hc, hc), :],
                send_sem=p1_send.at[c],
                recv_sem=p1_recv.at[c],
                device_id=(my_x, peer_y),
                device_id_type=pl.DeviceIdType.MESH,
            )
            d.start()
            p1.append(d)

        out_ref[pl.ds(my_y * m, m), :] = x_ref[:, pl.ds(my_y * n_half, n_half)]

        p2 = []
        for c in range(N_CHUNKS):
            r1 = pltpu.make_async_remote_copy(
                src_ref=x_ref.at[pl.ds(c * hc, hc), pl.ds(0, n_half)],
                dst_ref=out_ref.at[pl.ds(p1_base + c * hc, hc), :],
                send_sem=p1_send.at[c],
                recv_sem=p1_recv.at[c],
                device_id=(my_x, peer_y),
                device_id_type=pl.DeviceIdType.MESH,
            )
            r1.wait_recv()
            d2 = pltpu.make_async_remote_copy(
                src_ref=out_ref.at[pl.ds(p1_base + c * hc, hc), :],
                dst_ref=out_ref.at[pl.ds(p1_base + c * hc, hc), :],
                send_sem=p2_send.at[c],
                recv_sem=p2_recv.at[c],
                device_id=(peer_x, my_y),
                device_id_type=pl.DeviceIdType.MESH,
            )
            d2.start()
            p2.append(d2)

        for c in range(N_CHUNKS):
            r2 = pltpu.make_async_remote_copy(
                src_ref=x_ref.at[pl.ds(c * hc, hc), pl.ds(0, n_half)],
                dst_ref=out_ref.at[pl.ds(peer_y * m + peer_x * h + c * hc, hc), :],
                send_sem=p2_send.at[c],
                recv_sem=p2_recv.at[c],
                device_id=(peer_x, my_y),
                device_id_type=pl.DeviceIdType.MESH,
            )
            r2.wait_recv()

        for c in range(N_CHUNKS):
            p1[c].wait_send()
            p2[c].wait_send()

    return pl.pallas_call(
        body,
        out_shape=jax.ShapeDtypeStruct((2 * m, n_half), x.dtype),
        in_specs=[pl.BlockSpec(memory_space=pltpu.VMEM)],
        out_specs=pl.BlockSpec(memory_space=pltpu.VMEM),
        scratch_shapes=[
            pltpu.SemaphoreType.DMA((N_CHUNKS,)),
            pltpu.SemaphoreType.DMA((N_CHUNKS,)),
            pltpu.SemaphoreType.DMA((N_CHUNKS,)),
            pltpu.SemaphoreType.DMA((N_CHUNKS,)),
        ],
        compiler_params=pltpu.CompilerParams(collective_id=0),
    )(x)
